# baseline (speedup 1.0000x reference)
import functools

import jax
import jax.numpy as jnp
from jax import lax
from jax.experimental import pallas as pl
from jax.experimental.pallas import tpu as pltpu
from jax.experimental.pallas import tpu_sc as plsc

NUM_USERS = 60000
NUM_ITEMS = 40000
NN = NUM_USERS + NUM_ITEMS
EMB = 32
NUM_LAYERS = 3
NW = 32
NP = 100096
RPW = NP // NW
NE = 1600000
BLK = 128
SB = 256


def _make_layer_body(final):
    def body(*refs):
        if final:
            (table, col, row, val, starts, e0t, e1t, out,
             starts_v,
             colv0, colv1, rowv0, rowv1, valv0, valv1,
             gbuf0, gbuf1, sbuf0, sbuf1, dbuf0, dbuf1, shacc,
             lsem0, lsem1, gsem0, gsem1, ssem0, ssem1, zsem) = refs
        else:
            (table, col, row, val, starts, out,
             starts_v,
             colv0, colv1, rowv0, rowv1, valv0, valv1,
             gbuf0, gbuf1, sbuf0, sbuf1, dbuf0, dbuf1, shacc,
             lsem0, lsem1, gsem0, gsem1, ssem0, ssem1, zsem) = refs

        c = lax.axis_index("c")
        s = lax.axis_index("s")
        wid = s * 2 + c
        base = wid * RPW
        sbase = s * RPW

        colv = (colv0, colv1)
        rowv = (rowv0, rowv1)
        valv = (valv0, valv1)
        gbuf = (gbuf0, gbuf1)
        sbuf = (sbuf0, sbuf1)
        dbuf = (dbuf0, dbuf1)
        lsem = (lsem0, lsem1)
        gsem = (gsem0, gsem1)
        ssem = (ssem0, ssem1)

        pltpu.sync_copy(starts, starts_v)

        zeros = jnp.zeros((16,), jnp.float32)

        def zbody(i, carry):
            sbuf0[i, pl.ds(0, 16)] = zeros
            sbuf0[i, pl.ds(16, 16)] = zeros
            return carry

        lax.fori_loop(0, SB, zbody, 0)

        for q in range(RPW // SB):
            pltpu.async_copy(sbuf0, shacc.at[pl.ds(sbase + q * SB, SB)], zsem)
        pltpu.async_copy(sbuf0.at[pl.ds(0, RPW % SB)],
                         shacc.at[pl.ds(sbase + (RPW // SB) * SB, RPW % SB)],
                         zsem)
        for q in range(RPW // SB):
            pltpu.make_async_copy(sbuf0, shacc.at[pl.ds(0, SB)], zsem).wait()
        pltpu.make_async_copy(sbuf0.at[pl.ds(0, RPW % SB)],
                              shacc.at[pl.ds(0, RPW % SB)], zsem).wait()

        s_w = starts_v[pl.ds(wid, 16)][0]
        e_w = starts_v[pl.ds(wid + 1, 16)][0]
        k_lo = s_w // SB
        g_cnt = (e_w + SB - 1) // SB - k_lo
        gp = (g_cnt + 1) // 2

        def fire_linear(g, b):
            e0 = jnp.minimum((k_lo + g) * SB, NE - SB)
            pltpu.async_copy(col.at[pl.ds(e0, SB)], colv[b], lsem[b])
            pltpu.async_copy(row.at[pl.ds(e0, SB)], rowv[b], lsem[b])
            pltpu.async_copy(val.at[pl.ds(e0, SB)], valv[b], lsem[b])

        def wait_linear(b):
            pltpu.make_async_copy(col.at[pl.ds(0, SB)], colv[b], lsem[b]).wait()
            pltpu.make_async_copy(row.at[pl.ds(0, SB)], rowv[b], lsem[b]).wait()
            pltpu.make_async_copy(val.at[pl.ds(0, SB)], valv[b], lsem[b]).wait()

        def fire_gather(b):
            for h in range(SB // BLK):
                pltpu.async_copy(table.at[colv[b].at[pl.ds(h * BLK, BLK)]],
                                 gbuf[b].at[pl.ds(h * BLK, BLK)], gsem[b])

        def wait_gather(b):
            for h in range(SB // BLK):
                pltpu.make_async_copy(
                    table.at[colv[b].at[pl.ds(h * BLK, BLK)]],
                    gbuf[b].at[pl.ds(h * BLK, BLK)], gsem[b]).wait()

        hi_mask = jnp.int32(-65536)

        def widen(row32):
            y = plsc.bitcast(row32, jnp.int32)
            lo = plsc.bitcast(jnp.left_shift(y, 16), jnp.float32)
            hi = plsc.bitcast(jnp.bitwise_and(y, hi_mask), jnp.float32)
            return lo, hi

        def compute(b, e0):
            def chunk(jc, masked):
                j0 = jc * 16
                rv = rowv[b][pl.ds(j0, 16)]
                vv = valv[b][pl.ds(j0, 16)]
                dv = rv - base
                if masked:
                    okv = (dv >= 0) & (dv < RPW)
                    okv = okv & (e0 + j0 + lax.iota(jnp.int32, 16) < e_w)
                    dv = jnp.clip(dv, 0, RPW - 1)
                    vv = jnp.where(okv, vv, 0.0)
                h = jc // (BLK // 16)
                p = (jc % (BLK // 16)) * 16
                dbuf[b][h, pl.ds(p, 16)] = dv + sbase
                for t in range(16):
                    sv = vv[t]
                    lo, hi = widen(gbuf[b][j0 + t, pl.ds(0, 32)])
                    sbuf[b][j0 + t, pl.ds(0, 16)] = lo * sv
                    sbuf[b][j0 + t, pl.ds(16, 16)] = hi * sv

            boundary = (e0 < s_w) | (e0 + SB > e_w)

            @pl.when(boundary)
            def _():
                @plsc.parallel_loop(0, SB // 16, 1)
                def _(jc):
                    chunk(jc, True)

            @pl.when(jnp.logical_not(boundary))
            def _():
                @plsc.parallel_loop(0, SB // 16, 1)
                def _(jc):
                    chunk(jc, False)

        def fire_scatter(b):
            for h in range(SB // BLK):
                pltpu.async_copy(sbuf[b].at[pl.ds(h * BLK, BLK)],
                                 shacc.at[dbuf[b].at[h]], ssem[b], add=True)

        def wait_scatter(b):
            for h in range(SB // BLK):
                pltpu.make_async_copy(sbuf[b].at[pl.ds(h * BLK, BLK)],
                                      shacc.at[dbuf[b].at[h]], ssem[b]).wait()

        fire_linear(0, 0)
        fire_linear(1, 1)
        wait_linear(0)
        fire_gather(0)

        def pipe_body(g, b):
            wait_linear(1 - b)
            fire_gather(1 - b)
            wait_gather(b)

            @pl.when(g >= 2)
            def _():
                wait_scatter(b)

            compute(b, (k_lo + g) * SB)
            fire_scatter(b)
            fire_linear(g + 2, b)

        def pair(p, carry):
            pipe_body(2 * p, 0)
            pipe_body(2 * p + 1, 1)
            return carry

        lax.fori_loop(0, gp, pair, 0)

        wait_gather(0)
        wait_linear(1)

        @pl.when(gp > 0)
        def _():
            wait_scatter(0)
            wait_scatter(1)

        if not final:
            def pack_rows(r, rows):
                pltpu.sync_copy(shacc.at[pl.ds(sbase + r, rows)],
                                sbuf0.at[pl.ds(0, rows)])

                @plsc.parallel_loop(0, rows, 1)
                def _(i):
                    pk = plsc.pack(sbuf0[i, pl.ds(0, 16)],
                                   sbuf0[i, pl.ds(16, 16)],
                                   format=plsc.PackFormat.INTERLEAVED)
                    gbuf0[i, pl.ds(0, 32)] = pk

                pltpu.sync_copy(gbuf0.at[pl.ds(0, rows)],
                                out.at[pl.ds(base + r, rows)])

            def pack_loop(q, carry):
                pack_rows(q * SB, SB)
                return carry

            lax.fori_loop(0, RPW // SB, pack_loop, 0)
            pack_rows((RPW // SB) * SB, RPW % SB)
        else:
            def mean_rows(r, rows):
                pltpu.async_copy(e0t.at[pl.ds(base + r, rows)],
                                 sbuf0.at[pl.ds(0, rows)], lsem0)
                pltpu.async_copy(e1t.at[pl.ds(base + r, rows)],
                                 gbuf0.at[pl.ds(0, rows)], lsem1)
                pltpu.async_copy(table.at[pl.ds(base + r, rows)],
                                 gbuf1.at[pl.ds(0, rows)], gsem0)
                pltpu.make_async_copy(e0t.at[pl.ds(base + r, rows)],
                                      sbuf0.at[pl.ds(0, rows)], lsem0).wait()
                pltpu.make_async_copy(e1t.at[pl.ds(base + r, rows)],
                                      gbuf0.at[pl.ds(0, rows)], lsem1).wait()
                pltpu.make_async_copy(table.at[pl.ds(base + r, rows)],
                                      gbuf1.at[pl.ds(0, rows)], gsem0).wait()
                pltpu.sync_copy(shacc.at[pl.ds(sbase + r, rows)],
                                sbuf1.at[pl.ds(0, rows)])

                @plsc.parallel_loop(0, rows, 1)
                def _(i):
                    lo1, hi1 = widen(gbuf0[i, pl.ds(0, 32)])
                    lo2, hi2 = widen(gbuf1[i, pl.ds(0, 32)])
                    m_lo = (sbuf0[i, pl.ds(0, 16)] + lo1 + lo2
                            + sbuf1[i, pl.ds(0, 16)]) * 0.25
                    m_hi = (sbuf0[i, pl.ds(16, 16)] + hi1 + hi2
                            + sbuf1[i, pl.ds(16, 16)]) * 0.25
                    sbuf0[i, pl.ds(0, 16)] = m_lo
                    sbuf0[i, pl.ds(16, 16)] = m_hi

                pltpu.sync_copy(sbuf0.at[pl.ds(0, rows)],
                                out.at[pl.ds(base + r, rows)])

            def mean_loop(q, carry):
                mean_rows(q * SB, SB)
                return carry

            lax.fori_loop(0, RPW // SB, mean_loop, 0)
            mean_rows((RPW // SB) * SB, RPW % SB)

    return body


_SCRATCH = [
    pltpu.VMEM((48,), jnp.int32),
    pltpu.VMEM((SB,), jnp.int32),
    pltpu.VMEM((SB,), jnp.int32),
    pltpu.VMEM((SB,), jnp.int32),
    pltpu.VMEM((SB,), jnp.int32),
    pltpu.VMEM((SB,), jnp.float32),
    pltpu.VMEM((SB,), jnp.float32),
    pltpu.VMEM((SB, EMB), jnp.bfloat16),
    pltpu.VMEM((SB, EMB), jnp.bfloat16),
    pltpu.VMEM((SB, EMB), jnp.float32),
    pltpu.VMEM((SB, EMB), jnp.float32),
    pltpu.VMEM((SB // BLK, BLK), jnp.int32),
    pltpu.VMEM((SB // BLK, BLK), jnp.int32),
    pltpu.VMEM_SHARED((16 * RPW, EMB), jnp.float32),
    pltpu.SemaphoreType.DMA,
    pltpu.SemaphoreType.DMA,
    pltpu.SemaphoreType.DMA,
    pltpu.SemaphoreType.DMA,
    pltpu.SemaphoreType.DMA,
    pltpu.SemaphoreType.DMA,
    pltpu.SemaphoreType.DMA,
]


def _propagate(table, col, row, val, starts):
    mesh = plsc.VectorSubcoreMesh(core_axis_name="c", subcore_axis_name="s")
    fn = functools.partial(
        pl.kernel,
        mesh=mesh,
        out_type=jax.ShapeDtypeStruct((NP, EMB), jnp.bfloat16),
        compiler_params=pltpu.CompilerParams(use_tc_tiling_on_sc=False, needs_layout_passes=False),
        scratch_types=_SCRATCH,
    )(_make_layer_body(False))
    return fn(table, col, row, val, starts)


def _propagate_mean(table, col, row, val, starts, e0t, e1t):
    mesh = plsc.VectorSubcoreMesh(core_axis_name="c", subcore_axis_name="s")
    fn = functools.partial(
        pl.kernel,
        mesh=mesh,
        out_type=jax.ShapeDtypeStruct((NP, EMB), jnp.float32),
        compiler_params=pltpu.CompilerParams(use_tc_tiling_on_sc=False, needs_layout_passes=False),
        scratch_types=_SCRATCH,
    )(_make_layer_body(True))
    return fn(table, col, row, val, starts, e0t, e1t)


def kernel(user_emb, item_emb, adj_row, adj_col, adj_val):
    e0 = jnp.concatenate(
        [user_emb, item_emb, jnp.zeros((NP - NN, EMB), jnp.float32)], axis=0)
    e0b = e0.astype(jnp.bfloat16)
    e0p = jnp.concatenate([e0[:, 0::2], e0[:, 1::2]], axis=1)
    col = adj_col.astype(jnp.int32)
    row = adj_row.astype(jnp.int32)
    bounds = (jnp.arange(NW + 1, dtype=jnp.int32) * RPW).astype(adj_row.dtype)
    starts = jnp.searchsorted(adj_row, bounds, side="left").astype(jnp.int32)
    starts = jnp.concatenate([starts, jnp.zeros((15,), jnp.int32)])

    e1b = _propagate(e0b, col, row, adj_val, starts)
    e2b = _propagate(e1b, col, row, adj_val, starts)
    outp = _propagate_mean(e2b, col, row, adj_val, starts, e0p, e1b)

    out = jnp.stack([outp[:, :EMB // 2], outp[:, EMB // 2:]],
                    axis=2).reshape(NP, EMB)
    return out[:NUM_USERS], out[NUM_USERS:NN]


# speedup vs baseline: 1.2665x; 1.2665x over previous
import functools

import jax
import jax.numpy as jnp
from jax import lax
from jax.experimental import pallas as pl
from jax.experimental.pallas import tpu as pltpu
from jax.experimental.pallas import tpu_sc as plsc

NUM_USERS = 60000
NUM_ITEMS = 40000
NN = NUM_USERS + NUM_ITEMS
EMB = 32
NUM_LAYERS = 3
NW = 32
NP = 100096
RPW = NP // NW
NE = 1600000
BLK = 128
SB = 256


def _make_layer_body(final):
    def body(*refs):
        if final:
            (table, col, row, val, starts, e0t, e1t, out,
             starts_v,
             colv0, colv1, colv2, rowv0, rowv1, rowv2,
             valv0, valv1, valv2,
             gbuf0, gbuf1, gbuf2, dbuf0, dbuf1, dbuf2, shacc,
             lsem0, lsem1, lsem2, gsem0, gsem1, gsem2,
             ssem0, ssem1, ssem2, zsem) = refs
        else:
            (table, col, row, val, starts, out,
             starts_v,
             colv0, colv1, colv2, rowv0, rowv1, rowv2,
             valv0, valv1, valv2,
             gbuf0, gbuf1, gbuf2, dbuf0, dbuf1, dbuf2, shacc,
             lsem0, lsem1, lsem2, gsem0, gsem1, gsem2,
             ssem0, ssem1, ssem2, zsem) = refs

        c = lax.axis_index("c")
        s = lax.axis_index("s")
        wid = s * 2 + c
        base = wid * RPW
        sbase = s * RPW

        colv = (colv0, colv1, colv2)
        rowv = (rowv0, rowv1, rowv2)
        valv = (valv0, valv1, valv2)
        gbuf = (gbuf0, gbuf1, gbuf2)
        dbuf = (dbuf0, dbuf1, dbuf2)
        lsem = (lsem0, lsem1, lsem2)
        gsem = (gsem0, gsem1, gsem2)
        ssem = (ssem0, ssem1, ssem2)

        pltpu.sync_copy(starts, starts_v)

        zeros = jnp.zeros((16,), jnp.float32)

        def zbody(i, carry):
            gbuf0[i, pl.ds(0, 16)] = zeros
            gbuf0[i, pl.ds(16, 16)] = zeros
            return carry

        lax.fori_loop(0, SB, zbody, 0)

        for q in range(RPW // SB):
            pltpu.async_copy(gbuf0, shacc.at[pl.ds(sbase + q * SB, SB)], zsem)
        pltpu.async_copy(gbuf0.at[pl.ds(0, RPW % SB)],
                         shacc.at[pl.ds(sbase + (RPW // SB) * SB, RPW % SB)],
                         zsem)
        for q in range(RPW // SB):
            pltpu.make_async_copy(gbuf0, shacc.at[pl.ds(0, SB)], zsem).wait()
        pltpu.make_async_copy(gbuf0.at[pl.ds(0, RPW % SB)],
                              shacc.at[pl.ds(0, RPW % SB)], zsem).wait()

        s_w = starts_v[pl.ds(wid, 16)][0]
        e_w = starts_v[pl.ds(wid + 1, 16)][0]
        k_lo = s_w // SB
        g_cnt = (e_w + SB - 1) // SB - k_lo
        gp = (g_cnt + 2) // 3

        def fire_linear(g, b):
            e0 = jnp.minimum((k_lo + g) * SB, NE - SB)
            pltpu.async_copy(col.at[pl.ds(e0, SB)], colv[b], lsem[b])
            pltpu.async_copy(row.at[pl.ds(e0, SB)], rowv[b], lsem[b])
            pltpu.async_copy(val.at[pl.ds(e0, SB)], valv[b], lsem[b])

        def wait_linear(b):
            pltpu.make_async_copy(col.at[pl.ds(0, SB)], colv[b], lsem[b]).wait()
            pltpu.make_async_copy(row.at[pl.ds(0, SB)], rowv[b], lsem[b]).wait()
            pltpu.make_async_copy(val.at[pl.ds(0, SB)], valv[b], lsem[b]).wait()

        def fire_gather(b):
            for h in range(SB // BLK):
                pltpu.async_copy(table.at[colv[b].at[pl.ds(h * BLK, BLK)]],
                                 gbuf[b].at[pl.ds(h * BLK, BLK)], gsem[b])

        def wait_gather(b):
            for h in range(SB // BLK):
                pltpu.make_async_copy(
                    table.at[colv[b].at[pl.ds(h * BLK, BLK)]],
                    gbuf[b].at[pl.ds(h * BLK, BLK)], gsem[b]).wait()

        def compute(b, e0):
            def chunk(jc, masked):
                j0 = jc * 16
                rv = rowv[b][pl.ds(j0, 16)]
                vv = valv[b][pl.ds(j0, 16)]
                dv = rv - base
                if masked:
                    okv = (dv >= 0) & (dv < RPW)
                    okv = okv & (e0 + j0 + lax.iota(jnp.int32, 16) < e_w)
                    dv = jnp.clip(dv, 0, RPW - 1)
                    vv = jnp.where(okv, vv, 0.0)
                h = jc // (BLK // 16)
                p = (jc % (BLK // 16)) * 16
                dbuf[b][h, pl.ds(p, 16)] = dv + sbase
                for t in range(16):
                    sv = vv[t]
                    gbuf[b][j0 + t, pl.ds(0, 16)] = (
                        gbuf[b][j0 + t, pl.ds(0, 16)] * sv)
                    gbuf[b][j0 + t, pl.ds(16, 16)] = (
                        gbuf[b][j0 + t, pl.ds(16, 16)] * sv)

            boundary = (e0 < s_w) | (e0 + SB > e_w)

            @pl.when(boundary)
            def _():
                @plsc.parallel_loop(0, SB // 16, 1)
                def _(jc):
                    chunk(jc, True)

            @pl.when(jnp.logical_not(boundary))
            def _():
                @plsc.parallel_loop(0, SB // 16, 1)
                def _(jc):
                    chunk(jc, False)

        def fire_scatter(b):
            for h in range(SB // BLK):
                pltpu.async_copy(gbuf[b].at[pl.ds(h * BLK, BLK)],
                                 shacc.at[dbuf[b].at[h]], ssem[b], add=True)

        def wait_scatter(b):
            for h in range(SB // BLK):
                pltpu.make_async_copy(gbuf[b].at[pl.ds(h * BLK, BLK)],
                                      shacc.at[dbuf[b].at[h]], ssem[b]).wait()

        fire_linear(0, 0)
        fire_linear(1, 1)
        wait_linear(0)
        fire_gather(0)

        def pipe_body(g, b):
            nb = (b + 1) % 3
            wait_linear(nb)

            @pl.when(g >= 2)
            def _():
                wait_scatter(nb)

            fire_gather(nb)
            wait_gather(b)
            compute(b, (k_lo + g) * SB)
            fire_scatter(b)
            fire_linear(g + 2, (b + 2) % 3)

        def triple(p, carry):
            pipe_body(3 * p, 0)
            pipe_body(3 * p + 1, 1)
            pipe_body(3 * p + 2, 2)
            return carry

        lax.fori_loop(0, gp, triple, 0)

        wait_gather(0)
        wait_linear(1)

        @pl.when(gp > 0)
        def _():
            wait_scatter(1)
            wait_scatter(2)

        if not final:
            pltpu.sync_copy(shacc.at[pl.ds(sbase, RPW)],
                            out.at[pl.ds(base, RPW)])
        else:
            def mean_rows(r, rows):
                pltpu.async_copy(e0t.at[pl.ds(base + r, rows)],
                                 gbuf0.at[pl.ds(0, rows)], lsem0)
                pltpu.async_copy(e1t.at[pl.ds(base + r, rows)],
                                 gbuf0.at[pl.ds(BLK, rows)], lsem1)
                pltpu.async_copy(table.at[pl.ds(base + r, rows)],
                                 gbuf1.at[pl.ds(0, rows)], lsem2)
                pltpu.make_async_copy(e0t.at[pl.ds(base + r, rows)],
                                      gbuf0.at[pl.ds(0, rows)], lsem0).wait()
                pltpu.make_async_copy(e1t.at[pl.ds(base + r, rows)],
                                      gbuf0.at[pl.ds(BLK, rows)], lsem1).wait()
                pltpu.make_async_copy(table.at[pl.ds(base + r, rows)],
                                      gbuf1.at[pl.ds(0, rows)], lsem2).wait()
                pltpu.sync_copy(shacc.at[pl.ds(sbase + r, rows)],
                                gbuf1.at[pl.ds(BLK, rows)])

                @plsc.parallel_loop(0, rows, 1)
                def _(i):
                    for hh in (0, 16):
                        m = (gbuf0[i, pl.ds(hh, 16)]
                             + gbuf0[BLK + i, pl.ds(hh, 16)]
                             + gbuf1[i, pl.ds(hh, 16)]
                             + gbuf1[BLK + i, pl.ds(hh, 16)]) * 0.25
                        gbuf2[i, pl.ds(hh, 16)] = m

                pltpu.sync_copy(gbuf2.at[pl.ds(0, rows)],
                                out.at[pl.ds(base + r, rows)])

            def mean_loop(q, carry):
                mean_rows(q * BLK, BLK)
                return carry

            lax.fori_loop(0, RPW // BLK, mean_loop, 0)
            mean_rows((RPW // BLK) * BLK, RPW % BLK)

    return body


_SCRATCH = [
    pltpu.VMEM((48,), jnp.int32),
    pltpu.VMEM((SB,), jnp.int32),
    pltpu.VMEM((SB,), jnp.int32),
    pltpu.VMEM((SB,), jnp.int32),
    pltpu.VMEM((SB,), jnp.int32),
    pltpu.VMEM((SB,), jnp.int32),
    pltpu.VMEM((SB,), jnp.int32),
    pltpu.VMEM((SB,), jnp.float32),
    pltpu.VMEM((SB,), jnp.float32),
    pltpu.VMEM((SB,), jnp.float32),
    pltpu.VMEM((SB, EMB), jnp.float32),
    pltpu.VMEM((SB, EMB), jnp.float32),
    pltpu.VMEM((SB, EMB), jnp.float32),
    pltpu.VMEM((SB // BLK, BLK), jnp.int32),
    pltpu.VMEM((SB // BLK, BLK), jnp.int32),
    pltpu.VMEM((SB // BLK, BLK), jnp.int32),
    pltpu.VMEM_SHARED((16 * RPW, EMB), jnp.float32),
    pltpu.SemaphoreType.DMA,
    pltpu.SemaphoreType.DMA,
    pltpu.SemaphoreType.DMA,
    pltpu.SemaphoreType.DMA,
    pltpu.SemaphoreType.DMA,
    pltpu.SemaphoreType.DMA,
    pltpu.SemaphoreType.DMA,
    pltpu.SemaphoreType.DMA,
    pltpu.SemaphoreType.DMA,
    pltpu.SemaphoreType.DMA,
]


def _propagate(table, col, row, val, starts):
    mesh = plsc.VectorSubcoreMesh(core_axis_name="c", subcore_axis_name="s")
    fn = functools.partial(
        pl.kernel,
        mesh=mesh,
        out_type=jax.ShapeDtypeStruct((NP, EMB), jnp.float32),
        compiler_params=pltpu.CompilerParams(use_tc_tiling_on_sc=False),
        scratch_types=_SCRATCH,
    )(_make_layer_body(False))
    return fn(table, col, row, val, starts)


def _propagate_mean(table, col, row, val, starts, e0t, e1t):
    mesh = plsc.VectorSubcoreMesh(core_axis_name="c", subcore_axis_name="s")
    fn = functools.partial(
        pl.kernel,
        mesh=mesh,
        out_type=jax.ShapeDtypeStruct((NP, EMB), jnp.float32),
        compiler_params=pltpu.CompilerParams(use_tc_tiling_on_sc=False),
        scratch_types=_SCRATCH,
    )(_make_layer_body(True))
    return fn(table, col, row, val, starts, e0t, e1t)


def kernel(user_emb, item_emb, adj_row, adj_col, adj_val):
    e0 = jnp.concatenate(
        [user_emb, item_emb, jnp.zeros((NP - NN, EMB), jnp.float32)], axis=0)
    col = adj_col.astype(jnp.int32)
    row = adj_row.astype(jnp.int32)
    bounds = (jnp.arange(NW + 1, dtype=jnp.int32) * RPW).astype(adj_row.dtype)
    starts = jnp.searchsorted(adj_row, bounds, side="left").astype(jnp.int32)
    starts = jnp.concatenate([starts, jnp.zeros((15,), jnp.int32)])

    e1 = _propagate(e0, col, row, adj_val, starts)
    e2 = _propagate(e1, col, row, adj_val, starts)
    out = _propagate_mean(e2, col, row, adj_val, starts, e0, e1)

    return out[:NUM_USERS], out[NUM_USERS:NN]
